# baseline (speedup 1.0000x reference)
import jax
import jax.numpy as jnp
from jax import lax
from jax.experimental import pallas as pl
from jax.experimental.pallas import tpu as pltpu
from jax.experimental.pallas import tpu_sc as plsc

_B = 256
_N = 100000
_C = 1024
_NT = 16
_CT = _N // _NT
_LABPAD = 6272
_PADLAB = 1000
_SCAN_ITERS = _LABPAD // 16
_BATCH = 128
_MCOL = 6656


def _popcnt(m):
    return plsc.all_reduce_population_count(m)[0]


def _nca_sc_body(xt_ref, idx_ref, lab1_ref, lab2_ref, out_ref,
                 idx_v, y_v, head_v, next_v, lab_v, mcol_v, mhead_v,
                 buf0_v, buf1_v, pacc16_v, pacc_v, tmp_v, own16_v, ownbuf_v,
                 shared_p, shared_own, sem):
    sid = lax.axis_index("s")
    lanes = lax.iota(jnp.int32, 16)
    zero16 = jnp.zeros((16,), jnp.int32)
    zf16 = jnp.zeros((16,), jnp.float32)

    pltpu.sync_copy(idx_ref, idx_v)
    for j in range(_B // 128):
        pltpu.async_copy(lab1_ref.at[idx_v.at[pl.ds(j * 128, 128)]],
                         y_v.at[pl.ds(j * 128, 128)], sem).wait()

    def z_head(k, c):
        head_v[pl.ds(k * 16, 16)] = zero16
        return c
    lax.fori_loop(0, _C // 16, z_head, 0)

    lane0 = lanes == 0

    def chain(k, c):
        y16 = y_v[pl.ds(k * 16, 16)]
        for l in range(16):
            i = k * 16 + l
            yi = jnp.full((16,), y16[l], jnp.int32)
            hv = plsc.load_gather(head_v, [yi])
            plsc.store_scatter(next_v, [jnp.full((16,), i, jnp.int32)], hv,
                               mask=lane0)
            plsc.store_scatter(head_v, [yi],
                               jnp.full((16,), i + 1, jnp.int32), mask=lane0)
        return c
    lax.fori_loop(0, _B // 16, chain, 0)

    pltpu.sync_copy(lab2_ref.at[sid], lab_v)
    base = sid * _CT

    def scan1(k, mptr):
        loc = k * 16 + lanes
        valid = loc < _CT
        lab16 = lab_v[pl.ds(k * 16, 16)]
        h16 = plsc.load_gather(head_v, [lab16])
        m = (h16 > 0) & valid
        plsc.store_compressed(mcol_v.at[pl.ds(mptr, 16)], base + loc, mask=m)
        plsc.store_compressed(mhead_v.at[pl.ds(mptr, 16)], h16, mask=m)
        return mptr + _popcnt(m)
    mcount = lax.fori_loop(0, _SCAN_ITERS, scan1, jnp.int32(0))

    def z_tail(k, c):
        mcol_v[pl.ds(mcount + k * 16, 16)] = zero16
        return c
    lax.fori_loop(0, (3 * _BATCH) // 16, z_tail, 0)

    def z_p16(k, c):
        pacc16_v[pl.ds(k * 16, 16)] = zf16
        return c
    lax.fori_loop(0, (16 * _B) // 16, z_p16, 0)

    nb = (mcount + _BATCH - 1) // _BATCH

    def fire(b, buf):
        pltpu.async_copy(
            xt_ref.at[mcol_v.at[pl.ds(b * _BATCH, _BATCH)]], buf, sem)

    def wait(b, buf):
        pltpu.make_async_copy(
            xt_ref.at[mcol_v.at[pl.ds(b * _BATCH, _BATCH)]], buf, sem).wait()

    def process(b, buf):
        for v in range(_BATCH // 16):
            pos = b * _BATCH + v * 16 + lanes
            valid = pos < mcount
            mh16 = mhead_v[pl.ds(b * _BATCH + v * 16, 16)]
            cur0 = jnp.where(valid, mh16, 0)
            c_loc = v * 16 + lanes

            def w_cond(cur):
                return _popcnt(cur > 0) > 0

            def w_body(cur):
                m = cur > 0
                r = jnp.where(m, cur - 1, 0)
                val = plsc.load_gather(buf, [c_loc, r], mask=m)
                e = jnp.exp(jnp.where(m, val, 0.0))
                plsc.addupdate_scatter(pacc16_v, [lanes * _B + r], e, mask=m)
                return jnp.where(m, plsc.load_gather(next_v, [r]), 0)

            lax.while_loop(w_cond, w_body, cur0)

    fire(jnp.int32(0), buf0_v)

    def pair_cond(b):
        return b < nb

    def pair_body(b):
        fire(b + 1, buf1_v)
        wait(b, buf0_v)
        process(b, buf0_v)
        fire(b + 2, buf0_v)
        wait(b + 1, buf1_v)

        @pl.when(b + 1 < nb)
        def _():
            process(b + 1, buf1_v)
        return b + 2

    b_end = lax.while_loop(pair_cond, pair_body, jnp.int32(0))
    wait(b_end, buf0_v)

    def red16(k, c):
        acc = zf16
        for l in range(16):
            acc = acc + pacc16_v[pl.ds(l * _B + k * 16, 16)]
        pacc_v[pl.ds(k * 16, 16)] = acc
        return c
    lax.fori_loop(0, _B // 16, red16, 0)

    pltpu.async_copy(xt_ref.at[idx_v.at[pl.ds(sid * 16, 16)]],
                     ownbuf_v, sem).wait()
    own = plsc.load_gather(ownbuf_v, [lanes, sid * 16 + lanes])
    own16_v[pl.ds(0, 16)] = jnp.exp(own)
    pltpu.sync_copy(own16_v, shared_own.at[pl.ds(sid * 16, 16)])
    pltpu.sync_copy(pacc_v, shared_p.at[sid])
    plsc.subcore_barrier()

    @pl.when(sid == 0)
    def _():
        def red_tiles(w, c):
            pltpu.sync_copy(shared_p.at[w], tmp_v)

            def addv(k, c2):
                pacc_v[pl.ds(k * 16, 16)] = (pacc_v[pl.ds(k * 16, 16)]
                                             + tmp_v[pl.ds(k * 16, 16)])
                return c2
            lax.fori_loop(0, _B // 16, addv, 0)
            return c
        lax.fori_loop(1, _NT, red_tiles, 0)

        pltpu.sync_copy(shared_own, tmp_v)

        def sub_own(k, c):
            pacc_v[pl.ds(k * 16, 16)] = (pacc_v[pl.ds(k * 16, 16)]
                                         - tmp_v[pl.ds(k * 16, 16)])
            return c
        lax.fori_loop(0, _B // 16, sub_own, 0)
        pltpu.sync_copy(pacc_v, out_ref)


def kernel(x, features, indexes, labels):
    del features
    xt = x.T
    idx32 = indexes.astype(jnp.int32)
    lab32 = labels.astype(jnp.int32)
    lab2 = jnp.full((_NT, _LABPAD), _PADLAB, jnp.int32)
    lab2 = lab2.at[:, :_CT].set(lab32.reshape(_NT, _CT))

    mesh = plsc.VectorSubcoreMesh(core_axis_name="c", subcore_axis_name="s",
                                  num_cores=1)
    fn = pl.kernel(
        _nca_sc_body,
        out_type=jax.ShapeDtypeStruct((_B,), jnp.float32),
        mesh=mesh,
        compiler_params=pltpu.CompilerParams(needs_layout_passes=False),
        scratch_types=[
            pltpu.VMEM((_B,), jnp.int32),
            pltpu.VMEM((_B,), jnp.int32),
            pltpu.VMEM((_C,), jnp.int32),
            pltpu.VMEM((_B,), jnp.int32),
            pltpu.VMEM((_LABPAD,), jnp.int32),
            pltpu.VMEM((_MCOL,), jnp.int32),
            pltpu.VMEM((_MCOL,), jnp.int32),
            pltpu.VMEM((_BATCH, _B), jnp.float32),
            pltpu.VMEM((_BATCH, _B), jnp.float32),
            pltpu.VMEM((16 * _B,), jnp.float32),
            pltpu.VMEM((_B,), jnp.float32),
            pltpu.VMEM((_B,), jnp.float32),
            pltpu.VMEM((16,), jnp.float32),
            pltpu.VMEM((16, _B), jnp.float32),
            pltpu.VMEM_SHARED((_NT, _B), jnp.float32),
            pltpu.VMEM_SHARED((_B,), jnp.float32),
            pltpu.SemaphoreType.DMA,
        ],
    )
    return fn(xt, idx32, lab32, lab2)


# speedup vs baseline: 3.1599x; 3.1599x over previous
import jax
import jax.numpy as jnp
from jax import lax
from jax.experimental import pallas as pl
from jax.experimental.pallas import tpu as pltpu
from jax.experimental.pallas import tpu_sc as plsc

_B = 256
_N = 100000
_C = 1024
_NT = 16
_CT = _N // _NT
_LABPAD = 6272
_PADLAB = 1000
_SCAN_ITERS = _LABPAD // 16
_BATCH = 128
_MCOL = 6656


def _popcnt(m):
    return plsc.all_reduce_population_count(m)[0]


def _nca_sc_body(xt_ref, idx_ref, lab1_ref, lab2_ref, out_ref,
                 idx_v, y_v, head_v, next_v, lab_v, mcol_v, mhead_v,
                 buf0_v, buf1_v, pacc16_v, pacc_v, tmp_v, own16_v, ownbuf_v,
                 shared_p, shared_own, sem, sem0, sem1):
    sid = lax.axis_index("s")
    lanes = lax.iota(jnp.int32, 16)
    zero16 = jnp.zeros((16,), jnp.int32)
    zf16 = jnp.zeros((16,), jnp.float32)

    pltpu.sync_copy(idx_ref, idx_v)
    for j in range(_B // 128):
        pltpu.async_copy(lab1_ref.at[idx_v.at[pl.ds(j * 128, 128)]],
                         y_v.at[pl.ds(j * 128, 128)], sem).wait()

    def z_head(k, c):
        head_v[pl.ds(k * 16, 16)] = zero16
        return c
    lax.fori_loop(0, _C // 16, z_head, 0)

    lane0 = lanes == 0

    def chain(k, c):
        y16 = y_v[pl.ds(k * 16, 16)]
        for l in range(16):
            i = k * 16 + l
            yi = jnp.full((16,), y16[l], jnp.int32)
            hv = plsc.load_gather(head_v, [yi])
            plsc.store_scatter(next_v, [jnp.full((16,), i, jnp.int32)], hv,
                               mask=lane0)
            plsc.store_scatter(head_v, [yi],
                               jnp.full((16,), i + 1, jnp.int32), mask=lane0)
        return c
    lax.fori_loop(0, _B // 16, chain, 0)

    pltpu.sync_copy(lab2_ref.at[sid], lab_v)
    base = sid * _CT

    def scan1(k, mptr):
        loc = k * 16 + lanes
        valid = loc < _CT
        lab16 = lab_v[pl.ds(k * 16, 16)]
        h16 = plsc.load_gather(head_v, [lab16])
        m = (h16 > 0) & valid
        plsc.store_compressed(mcol_v.at[pl.ds(mptr, 16)], base + loc, mask=m)
        plsc.store_compressed(mhead_v.at[pl.ds(mptr, 16)], h16, mask=m)
        return mptr + _popcnt(m)
    mcount = lax.fori_loop(0, _SCAN_ITERS, scan1, jnp.int32(0))

    def z_tail(k, c):
        mcol_v[pl.ds(mcount + k * 16, 16)] = sid * _BATCH + k * 16 + lanes
        return c
    lax.fori_loop(0, _BATCH // 16, z_tail, 0)

    def z_p16(k, c):
        pacc16_v[pl.ds(k * 16, 16)] = zf16
        return c
    lax.fori_loop(0, (16 * _B) // 16, z_p16, 0)

    nb = (mcount + _BATCH - 1) // _BATCH

    def fire(b, buf, bsem):
        pltpu.async_copy(
            xt_ref.at[mcol_v.at[pl.ds(b * _BATCH, _BATCH)]], buf, bsem)

    def wait(b, buf, bsem):
        pltpu.make_async_copy(
            xt_ref.at[mcol_v.at[pl.ds(b * _BATCH, _BATCH)]], buf,
            bsem).wait()

    def process(b, buf):
        for v in range(_BATCH // 16):
            pos = b * _BATCH + v * 16 + lanes
            valid = pos < mcount
            mh16 = mhead_v[pl.ds(b * _BATCH + v * 16, 16)]
            cur0 = jnp.where(valid, mh16, 0)
            c_loc = v * 16 + lanes

            def w_cond(cur):
                return _popcnt(cur > 0) > 0

            def w_body(cur):
                m = cur > 0
                r = jnp.where(m, cur - 1, 0)
                val = plsc.load_gather(buf, [c_loc, r], mask=m)
                e = jnp.exp(jnp.where(m, val, 0.0))
                plsc.addupdate_scatter(pacc16_v, [lanes * _B + r], e, mask=m)
                return jnp.where(m, plsc.load_gather(next_v, [r]), 0)

            lax.while_loop(w_cond, w_body, cur0)

    @pl.when(nb > 0)
    def _():
        fire(jnp.int32(0), buf0_v, sem0)

    def pair_cond(b):
        return b < nb

    def pair_body(b):
        @pl.when(b + 1 < nb)
        def _():
            fire(b + 1, buf1_v, sem1)
        wait(b, buf0_v, sem0)
        process(b, buf0_v)

        @pl.when(b + 2 < nb)
        def _():
            fire(b + 2, buf0_v, sem0)

        @pl.when(b + 1 < nb)
        def _():
            wait(b + 1, buf1_v, sem1)
            process(b + 1, buf1_v)
        return b + 2

    lax.while_loop(pair_cond, pair_body, jnp.int32(0))

    def red16(k, c):
        acc = zf16
        for l in range(16):
            acc = acc + pacc16_v[pl.ds(l * _B + k * 16, 16)]
        pacc_v[pl.ds(k * 16, 16)] = acc
        return c
    lax.fori_loop(0, _B // 16, red16, 0)

    pltpu.async_copy(xt_ref.at[idx_v.at[pl.ds(sid * 16, 16)]],
                     ownbuf_v, sem).wait()
    own = plsc.load_gather(ownbuf_v, [lanes, sid * 16 + lanes])
    own16_v[pl.ds(0, 16)] = jnp.exp(own)
    pltpu.sync_copy(own16_v, shared_own.at[pl.ds(sid * 16, 16)])
    pltpu.sync_copy(pacc_v, shared_p.at[sid])
    plsc.subcore_barrier()

    @pl.when(sid == 0)
    def _():
        def red_tiles(w, c):
            pltpu.sync_copy(shared_p.at[w], tmp_v)

            def addv(k, c2):
                pacc_v[pl.ds(k * 16, 16)] = (pacc_v[pl.ds(k * 16, 16)]
                                             + tmp_v[pl.ds(k * 16, 16)])
                return c2
            lax.fori_loop(0, _B // 16, addv, 0)
            return c
        lax.fori_loop(1, _NT, red_tiles, 0)

        pltpu.sync_copy(shared_own, tmp_v)

        def sub_own(k, c):
            pacc_v[pl.ds(k * 16, 16)] = (pacc_v[pl.ds(k * 16, 16)]
                                         - tmp_v[pl.ds(k * 16, 16)])
            return c
        lax.fori_loop(0, _B // 16, sub_own, 0)
        pltpu.sync_copy(pacc_v, out_ref)


def kernel(x, features, indexes, labels):
    del features
    xt = x.T
    idx32 = indexes.astype(jnp.int32)
    lab32 = labels.astype(jnp.int32)
    lab2 = jnp.full((_NT, _LABPAD), _PADLAB, jnp.int32)
    lab2 = lab2.at[:, :_CT].set(lab32.reshape(_NT, _CT))

    mesh = plsc.VectorSubcoreMesh(core_axis_name="c", subcore_axis_name="s",
                                  num_cores=1)
    fn = pl.kernel(
        _nca_sc_body,
        out_type=jax.ShapeDtypeStruct((_B,), jnp.float32),
        mesh=mesh,
        compiler_params=pltpu.CompilerParams(needs_layout_passes=False),
        scratch_types=[
            pltpu.VMEM((_B,), jnp.int32),
            pltpu.VMEM((_B,), jnp.int32),
            pltpu.VMEM((_C,), jnp.int32),
            pltpu.VMEM((_B,), jnp.int32),
            pltpu.VMEM((_LABPAD,), jnp.int32),
            pltpu.VMEM((_MCOL,), jnp.int32),
            pltpu.VMEM((_MCOL,), jnp.int32),
            pltpu.VMEM((_BATCH, _B), jnp.float32),
            pltpu.VMEM((_BATCH, _B), jnp.float32),
            pltpu.VMEM((16 * _B,), jnp.float32),
            pltpu.VMEM((_B,), jnp.float32),
            pltpu.VMEM((_B,), jnp.float32),
            pltpu.VMEM((16,), jnp.float32),
            pltpu.VMEM((16, _B), jnp.float32),
            pltpu.VMEM_SHARED((_NT, _B), jnp.float32),
            pltpu.VMEM_SHARED((_B,), jnp.float32),
            pltpu.SemaphoreType.DMA,
            pltpu.SemaphoreType.DMA,
            pltpu.SemaphoreType.DMA,
        ],
    )
    return fn(xt, idx32, lab32, lab2)


# speedup vs baseline: 3.7625x; 1.1907x over previous
import jax
import jax.numpy as jnp
from jax import lax
from jax.experimental import pallas as pl
from jax.experimental.pallas import tpu as pltpu
from jax.experimental.pallas import tpu_sc as plsc

_B = 256
_N = 100000
_C = 1024
_NC = 2
_NS = 16
_NW = _NC * _NS
_CT = _N // _NW
_LABPAD = 3200
_PADLAB = 1000
_SCAN_ITERS = _LABPAD // 16
_BATCH = 128
_MCOL = 3328
_RPT = _B // _NS


def _popcnt(m):
    return plsc.all_reduce_population_count(m)[0]


def _nca_sc_body(xt_ref, idx_ref, lab1_ref, lab2_ref, out_ref,
                 idx_v, y_v, head_v, next_v, lab_v, mcol_v, mhead_v,
                 buf0_v, buf1_v, pacc16_v, pacc_v, tmp_v, own16_v, ownbuf_v,
                 shared_p, shared_own, sem, sem0, sem1):
    sid = lax.axis_index("s")
    cid = lax.axis_index("c")
    wid = cid * _NS + sid
    lanes = lax.iota(jnp.int32, 16)
    zero16 = jnp.zeros((16,), jnp.int32)
    zf16 = jnp.zeros((16,), jnp.float32)

    pltpu.sync_copy(idx_ref, idx_v)
    for j in range(_B // 128):
        pltpu.async_copy(lab1_ref.at[idx_v.at[pl.ds(j * 128, 128)]],
                         y_v.at[pl.ds(j * 128, 128)], sem).wait()

    def z_head(k, c):
        head_v[pl.ds(k * 16, 16)] = zero16
        return c
    lax.fori_loop(0, _C // 16, z_head, 0)

    lane0 = lanes == 0

    def chain(k, c):
        y16 = y_v[pl.ds(k * 16, 16)]
        for l in range(16):
            i = k * 16 + l
            yi = jnp.full((16,), y16[l], jnp.int32)
            hv = plsc.load_gather(head_v, [yi])
            plsc.store_scatter(next_v, [jnp.full((16,), i, jnp.int32)], hv,
                               mask=lane0)
            plsc.store_scatter(head_v, [yi],
                               jnp.full((16,), i + 1, jnp.int32), mask=lane0)
        return c
    lax.fori_loop(0, _B // 16, chain, 0)

    pltpu.sync_copy(lab2_ref.at[wid], lab_v)
    base = wid * _CT

    def scan1(k, mptr):
        loc = k * 16 + lanes
        valid = loc < _CT
        lab16 = lab_v[pl.ds(k * 16, 16)]
        h16 = plsc.load_gather(head_v, [lab16])
        m = (h16 > 0) & valid
        plsc.store_compressed(mcol_v.at[pl.ds(mptr, 16)], base + loc, mask=m)
        plsc.store_compressed(mhead_v.at[pl.ds(mptr, 16)], h16, mask=m)
        return mptr + _popcnt(m)
    mcount = lax.fori_loop(0, _SCAN_ITERS, scan1, jnp.int32(0))

    def z_tail(k, c):
        mcol_v[pl.ds(mcount + k * 16, 16)] = wid * _BATCH + k * 16 + lanes
        return c
    lax.fori_loop(0, _BATCH // 16, z_tail, 0)

    def z_p16(k, c):
        pacc16_v[pl.ds(k * 16, 16)] = zf16
        return c
    lax.fori_loop(0, (16 * _B) // 16, z_p16, 0)

    nb = (mcount + _BATCH - 1) // _BATCH

    def fire(b, buf, bsem):
        pltpu.async_copy(
            xt_ref.at[mcol_v.at[pl.ds(b * _BATCH, _BATCH)]], buf, bsem)

    def wait(b, buf, bsem):
        pltpu.make_async_copy(
            xt_ref.at[mcol_v.at[pl.ds(b * _BATCH, _BATCH)]], buf,
            bsem).wait()

    def process(b, buf):
        for v in range(_BATCH // 16):
            pos = b * _BATCH + v * 16 + lanes
            valid = pos < mcount
            mh16 = mhead_v[pl.ds(b * _BATCH + v * 16, 16)]
            cur0 = jnp.where(valid, mh16, 0)
            c_loc = v * 16 + lanes

            def w_cond(cur):
                return _popcnt(cur > 0) > 0

            def w_body(cur):
                m = cur > 0
                r = jnp.where(m, cur - 1, 0)
                val = plsc.load_gather(buf, [c_loc, r], mask=m)
                e = jnp.exp(jnp.where(m, val, 0.0))
                plsc.addupdate_scatter(pacc16_v, [lanes * _B + r], e, mask=m)
                return jnp.where(m, plsc.load_gather(next_v, [r]), 0)

            lax.while_loop(w_cond, w_body, cur0)

    @pl.when(nb > 0)
    def _():
        fire(jnp.int32(0), buf0_v, sem0)

    def pair_cond(b):
        return b < nb

    def pair_body(b):
        @pl.when(b + 1 < nb)
        def _():
            fire(b + 1, buf1_v, sem1)
        wait(b, buf0_v, sem0)
        process(b, buf0_v)

        @pl.when(b + 2 < nb)
        def _():
            fire(b + 2, buf0_v, sem0)

        @pl.when(b + 1 < nb)
        def _():
            wait(b + 1, buf1_v, sem1)
            process(b + 1, buf1_v)
        return b + 2

    lax.while_loop(pair_cond, pair_body, jnp.int32(0))

    def red16(k, c):
        acc = zf16
        for l in range(16):
            acc = acc + pacc16_v[pl.ds(l * _B + k * 16, 16)]
        pacc_v[pl.ds(k * 16, 16)] = acc
        return c
    lax.fori_loop(0, _B // 16, red16, 0)

    @pl.when(cid == 0)
    def _():
        pltpu.async_copy(xt_ref.at[idx_v.at[pl.ds(sid * 16, 16)]],
                         ownbuf_v, sem).wait()
        own = plsc.load_gather(ownbuf_v, [lanes, sid * 16 + lanes])
        own16_v[pl.ds(0, 16)] = jnp.exp(own)
        pltpu.sync_copy(own16_v, shared_own.at[pl.ds(sid * 16, 16)])
    pltpu.sync_copy(pacc_v, shared_p.at[sid])
    plsc.subcore_barrier()

    @pl.when(sid == 0)
    def _():
        def red_tiles(w, c):
            pltpu.sync_copy(shared_p.at[w], tmp_v)

            def addv(k, c2):
                pacc_v[pl.ds(k * 16, 16)] = (pacc_v[pl.ds(k * 16, 16)]
                                             + tmp_v[pl.ds(k * 16, 16)])
                return c2
            lax.fori_loop(0, _B // 16, addv, 0)
            return c
        lax.fori_loop(1, _NS, red_tiles, 0)

        @pl.when(cid == 0)
        def _():
            pltpu.sync_copy(shared_own, tmp_v)

            def sub_own(k, c):
                pacc_v[pl.ds(k * 16, 16)] = (pacc_v[pl.ds(k * 16, 16)]
                                             - tmp_v[pl.ds(k * 16, 16)])
                return c
            lax.fori_loop(0, _B // 16, sub_own, 0)
        pltpu.sync_copy(pacc_v, out_ref.at[cid])


def kernel(x, features, indexes, labels):
    del features
    xt = x.T
    idx32 = indexes.astype(jnp.int32)
    lab32 = labels.astype(jnp.int32)
    lab2 = jnp.full((_NW, _LABPAD), _PADLAB, jnp.int32)
    lab2 = lab2.at[:, :_CT].set(lab32.reshape(_NW, _CT))

    mesh = plsc.VectorSubcoreMesh(core_axis_name="c", subcore_axis_name="s")
    fn = pl.kernel(
        _nca_sc_body,
        out_type=jax.ShapeDtypeStruct((_NC, _B), jnp.float32),
        mesh=mesh,
        compiler_params=pltpu.CompilerParams(needs_layout_passes=False),
        scratch_types=[
            pltpu.VMEM((_B,), jnp.int32),
            pltpu.VMEM((_B,), jnp.int32),
            pltpu.VMEM((_C,), jnp.int32),
            pltpu.VMEM((_B,), jnp.int32),
            pltpu.VMEM((_LABPAD,), jnp.int32),
            pltpu.VMEM((_MCOL,), jnp.int32),
            pltpu.VMEM((_MCOL,), jnp.int32),
            pltpu.VMEM((_BATCH, _B), jnp.float32),
            pltpu.VMEM((_BATCH, _B), jnp.float32),
            pltpu.VMEM((16 * _B,), jnp.float32),
            pltpu.VMEM((_B,), jnp.float32),
            pltpu.VMEM((_B,), jnp.float32),
            pltpu.VMEM((16,), jnp.float32),
            pltpu.VMEM((16, _B), jnp.float32),
            pltpu.VMEM_SHARED((_NS, _B), jnp.float32),
            pltpu.VMEM_SHARED((_B,), jnp.float32),
            pltpu.SemaphoreType.DMA,
            pltpu.SemaphoreType.DMA,
            pltpu.SemaphoreType.DMA,
        ],
    )
    partial = fn(xt, idx32, lab32, lab2)
    return partial[0] + partial[1]


# speedup vs baseline: 3.9360x; 1.0461x over previous
import jax
import jax.numpy as jnp
from jax import lax
from jax.experimental import pallas as pl
from jax.experimental.pallas import tpu as pltpu
from jax.experimental.pallas import tpu_sc as plsc

_B = 256
_N = 100000
_C = 1024
_NC = 2
_NS = 16
_NW = _NC * _NS
_CT = _N // _NW
_LABPAD = 3200
_PADLAB = 1000
_SCAN_ITERS = _LABPAD // 16
_BATCH = 128
_MCOL = 3328
_RPT = _B // _NS


def _popcnt(m):
    return plsc.all_reduce_population_count(m)[0]


def _nca_sc_body(xt_ref, idx_ref, lab1_ref, lab2_ref, out_ref,
                 idx_v, y_v, head_v, next_v, lab_v, mcol_v, mhead_v,
                 buf0_v, buf1_v, pacc16_v, pacc_v, tmp16_v, own16_v,
                 ownbuf_v, shared_p, sem, sem0, sem1, sem2, sem3):
    sid = lax.axis_index("s")
    cid = lax.axis_index("c")
    wid = cid * _NS + sid
    lanes = lax.iota(jnp.int32, 16)
    zero16 = jnp.zeros((16,), jnp.int32)
    zf16 = jnp.zeros((16,), jnp.float32)

    lab2_dma = pltpu.async_copy(lab2_ref.at[wid], lab_v, sem2)
    pltpu.sync_copy(idx_ref, idx_v)

    @pl.when(cid == 0)
    def _():
        pltpu.async_copy(xt_ref.at[idx_v.at[pl.ds(sid * 16, 16)]],
                         ownbuf_v, sem3)
    for j in range(_B // 128):
        pltpu.async_copy(lab1_ref.at[idx_v.at[pl.ds(j * 128, 128)]],
                         y_v.at[pl.ds(j * 128, 128)], sem).wait()

    def z_head(k, c):
        head_v[pl.ds(k * 16, 16)] = zero16
        return c
    lax.fori_loop(0, _C // 16, z_head, 0)

    lane0 = lanes == 0

    def chain(k, c):
        y16 = y_v[pl.ds(k * 16, 16)]
        for l in range(16):
            i = k * 16 + l
            yi = jnp.full((16,), y16[l], jnp.int32)
            hv = plsc.load_gather(head_v, [yi])
            plsc.store_scatter(next_v, [jnp.full((16,), i, jnp.int32)], hv,
                               mask=lane0)
            plsc.store_scatter(head_v, [yi],
                               jnp.full((16,), i + 1, jnp.int32), mask=lane0)
        return c
    lax.fori_loop(0, _B // 16, chain, 0)

    lab2_dma.wait()
    base = wid * _CT

    def scan1(k, mptr):
        loc = k * 16 + lanes
        valid = loc < _CT
        lab16 = lab_v[pl.ds(k * 16, 16)]
        h16 = plsc.load_gather(head_v, [lab16])
        m = (h16 > 0) & valid
        plsc.store_compressed(mcol_v.at[pl.ds(mptr, 16)], base + loc, mask=m)
        plsc.store_compressed(mhead_v.at[pl.ds(mptr, 16)], h16, mask=m)
        return mptr + _popcnt(m)
    mcount = lax.fori_loop(0, _SCAN_ITERS, scan1, jnp.int32(0))

    def z_tail(k, c):
        mcol_v[pl.ds(mcount + k * 16, 16)] = wid * _BATCH + k * 16 + lanes
        return c
    lax.fori_loop(0, _BATCH // 16, z_tail, 0)

    def z_p16(k, c):
        pacc16_v[pl.ds(k * 16, 16)] = zf16
        return c
    lax.fori_loop(0, (16 * _B) // 16, z_p16, 0)

    nb = (mcount + _BATCH - 1) // _BATCH

    def fire(b, buf, bsem):
        pltpu.async_copy(
            xt_ref.at[mcol_v.at[pl.ds(b * _BATCH, _BATCH)]], buf, bsem)

    def wait(b, buf, bsem):
        pltpu.make_async_copy(
            xt_ref.at[mcol_v.at[pl.ds(b * _BATCH, _BATCH)]], buf,
            bsem).wait()

    def process(b, buf):
        for v in range(_BATCH // 16):
            pos = b * _BATCH + v * 16 + lanes
            valid = pos < mcount
            mh16 = mhead_v[pl.ds(b * _BATCH + v * 16, 16)]
            cur0 = jnp.where(valid, mh16, 0)
            c_loc = v * 16 + lanes

            def w_cond(cur):
                return _popcnt(cur > 0) > 0

            def w_body(cur):
                m = cur > 0
                r = jnp.where(m, cur - 1, 0)
                val = plsc.load_gather(buf, [c_loc, r], mask=m)
                e = jnp.exp(jnp.where(m, val, 0.0))
                plsc.addupdate_scatter(pacc16_v, [lanes * _B + r], e, mask=m)
                return jnp.where(m, plsc.load_gather(next_v, [r]), 0)

            lax.while_loop(w_cond, w_body, cur0)

    @pl.when(nb > 0)
    def _():
        fire(jnp.int32(0), buf0_v, sem0)

    def pair_cond(b):
        return b < nb

    def pair_body(b):
        @pl.when(b + 1 < nb)
        def _():
            fire(b + 1, buf1_v, sem1)
        wait(b, buf0_v, sem0)
        process(b, buf0_v)

        @pl.when(b + 2 < nb)
        def _():
            fire(b + 2, buf0_v, sem0)

        @pl.when(b + 1 < nb)
        def _():
            wait(b + 1, buf1_v, sem1)
            process(b + 1, buf1_v)
        return b + 2

    lax.while_loop(pair_cond, pair_body, jnp.int32(0))

    def red16(k, c):
        acc = zf16
        for l in range(16):
            acc = acc + pacc16_v[pl.ds(l * _B + k * 16, 16)]
        pacc_v[pl.ds(k * 16, 16)] = acc
        return c
    lax.fori_loop(0, _B // 16, red16, 0)

    @pl.when(cid == 0)
    def _():
        pltpu.make_async_copy(xt_ref.at[idx_v.at[pl.ds(sid * 16, 16)]],
                              ownbuf_v, sem3).wait()
        own = plsc.load_gather(ownbuf_v, [lanes, sid * 16 + lanes])
        pacc_v[pl.ds(sid * 16, 16)] = (pacc_v[pl.ds(sid * 16, 16)]
                                       - jnp.exp(own))
    pltpu.sync_copy(pacc_v, shared_p.at[sid])
    plsc.subcore_barrier()

    acc = zf16
    for w in range(_NS):
        pltpu.sync_copy(shared_p.at[w, pl.ds(sid * 16, 16)], tmp16_v)
        acc = acc + tmp16_v[pl.ds(0, 16)]
    own16_v[pl.ds(0, 16)] = acc
    pltpu.sync_copy(own16_v, out_ref.at[pl.ds(cid * _B + sid * 16, 16)])


def kernel(x, features, indexes, labels):
    del features
    xt = x.T
    idx32 = indexes.astype(jnp.int32)
    lab32 = labels.astype(jnp.int32)
    lab2 = jnp.full((_NW, _LABPAD), _PADLAB, jnp.int32)
    lab2 = lab2.at[:, :_CT].set(lab32.reshape(_NW, _CT))

    mesh = plsc.VectorSubcoreMesh(core_axis_name="c", subcore_axis_name="s")
    fn = pl.kernel(
        _nca_sc_body,
        out_type=jax.ShapeDtypeStruct((_NC * _B,), jnp.float32),
        mesh=mesh,
        compiler_params=pltpu.CompilerParams(needs_layout_passes=False),
        scratch_types=[
            pltpu.VMEM((_B,), jnp.int32),
            pltpu.VMEM((_B,), jnp.int32),
            pltpu.VMEM((_C,), jnp.int32),
            pltpu.VMEM((_B,), jnp.int32),
            pltpu.VMEM((_LABPAD,), jnp.int32),
            pltpu.VMEM((_MCOL,), jnp.int32),
            pltpu.VMEM((_MCOL,), jnp.int32),
            pltpu.VMEM((_BATCH, _B), jnp.float32),
            pltpu.VMEM((_BATCH, _B), jnp.float32),
            pltpu.VMEM((16 * _B,), jnp.float32),
            pltpu.VMEM((_B,), jnp.float32),
            pltpu.VMEM((16,), jnp.float32),
            pltpu.VMEM((16,), jnp.float32),
            pltpu.VMEM((16, _B), jnp.float32),
            pltpu.VMEM_SHARED((_NS, _B), jnp.float32),
            pltpu.SemaphoreType.DMA,
            pltpu.SemaphoreType.DMA,
            pltpu.SemaphoreType.DMA,
            pltpu.SemaphoreType.DMA,
            pltpu.SemaphoreType.DMA,
        ],
    )
    partial = fn(xt, idx32, lab32, lab2)
    return partial[:_B] + partial[_B:]
